# Initial kernel scaffold; baseline (speedup 1.0000x reference)
import functools

import jax
import jax.numpy as jnp
from jax import lax
from jax.experimental import pallas as pl
from jax.experimental.pallas import tpu as pltpu
from jax.experimental.pallas import tpu_sc as plsc

N = 10000
E = 320000
D = 128
L = 16

NC = 2
NS = 16
NW = NC * NS
EDGES_PER_W = E // NW
CHUNK = 80
NCHUNKS = EDGES_PER_W // CHUNK
N_PAD = 10240
ROWS_PER_SUB = N_PAD // NS
ZROWS = 80


def _sc_side(ego, row3, col, vals):
    mesh = plsc.VectorSubcoreMesh(core_axis_name="c", subcore_axis_name="s")

    @functools.partial(
        pl.kernel,
        out_type=jax.ShapeDtypeStruct((NC, N_PAD, D), jnp.float32),
        mesh=mesh,
        scratch_types=[
            pltpu.VMEM((NCHUNKS, CHUNK), jnp.int32),
            pltpu.VMEM((EDGES_PER_W,), jnp.int32),
            pltpu.VMEM((EDGES_PER_W,), jnp.float32),
            pltpu.VMEM((CHUNK, D), jnp.float32),
            pltpu.VMEM((ZROWS, D), jnp.float32),
            pltpu.VMEM_SHARED((N_PAD, D), jnp.float32),
            pltpu.SemaphoreType.DMA,
        ],
    )
    def body(ego_hbm, row_hbm, col_hbm, val_hbm, out_hbm,
             row_v, col_v, val_v, rows_v, zero_v, acc_sh, sem):
        core = lax.axis_index("c")
        sid = lax.axis_index("s")
        wid = core * NS + sid
        base = wid * EDGES_PER_W

        zvec = jnp.zeros((L,), jnp.float32)

        @pl.loop(0, ZROWS)
        def _(i):
            for j in range(D // L):
                zero_v[i, pl.ds(j * L, L)] = zvec

        @pl.loop(0, ROWS_PER_SUB // ZROWS)
        def _(k):
            pltpu.sync_copy(
                zero_v,
                acc_sh.at[pl.ds(sid * ROWS_PER_SUB + k * ZROWS, ZROWS)])

        pltpu.sync_copy(row_hbm.at[wid], row_v)
        pltpu.sync_copy(col_hbm.at[pl.ds(base, EDGES_PER_W)], col_v)
        pltpu.sync_copy(val_hbm.at[pl.ds(base, EDGES_PER_W)], val_v)

        plsc.subcore_barrier()

        @pl.loop(0, NCHUNKS)
        def _(c):
            pltpu.async_copy(
                ego_hbm.at[col_v.at[pl.ds(c * CHUNK, CHUNK)]], rows_v,
                sem).wait()

            @pl.loop(0, CHUNK)
            def _(i):
                idx = jnp.full((L,), c * CHUNK + i, jnp.int32)
                vv = plsc.load_gather(val_v, [idx])
                for j in range(D // L):
                    sl = (i, pl.ds(j * L, L))
                    rows_v[sl] = rows_v[sl] * vv

            pltpu.sync_copy(rows_v, acc_sh.at[row_v.at[c]], add=True)

        plsc.subcore_barrier()

        pltpu.sync_copy(
            acc_sh.at[pl.ds(sid * ROWS_PER_SUB, ROWS_PER_SUB)],
            out_hbm.at[core, pl.ds(sid * ROWS_PER_SUB, ROWS_PER_SUB)])

    return body(ego, row3, col, vals)


def _tc_body(ego_b, s_b0, s_b1, w_b, bias_b, out_b):
    hi = ego_b[...] + s_b0[...][0] + s_b1[...][0]
    acc = lax.dot_general(hi, w_b[...], (((1,), (1,)), ((), ())),
                          preferred_element_type=jnp.float32)
    acc = acc + bias_b[...]
    out_b[...] = jnp.where(acc >= 0, acc, 0.01 * acc)


def _tc_linear(ego, partials, W, bias2d):
    blk = 1000
    return pl.pallas_call(
        _tc_body,
        grid=(N // blk,),
        in_specs=[
            pl.BlockSpec((blk, D), lambda i: (i, 0)),
            pl.BlockSpec((1, blk, D), lambda i: (0, i, 0)),
            pl.BlockSpec((1, blk, D), lambda i: (1, i, 0)),
            pl.BlockSpec((D, D), lambda i: (0, 0)),
            pl.BlockSpec((1, D), lambda i: (0, 0)),
        ],
        out_specs=pl.BlockSpec((blk, D), lambda i: (i, 0)),
        out_shape=jax.ShapeDtypeStruct((N, D), jnp.float32),
    )(ego, partials, partials, W, bias2d)


def kernel(ego_embeddings, edge_index, edge_vals, h0, W, b, lamda, alpha, l):
    row = edge_index[0].reshape(NW, NCHUNKS, CHUNK)
    col = edge_index[1]
    partials = _sc_side(ego_embeddings, row, col, edge_vals)
    return _tc_linear(ego_embeddings, partials, W, b.reshape(1, D))


# speedup vs baseline: 5.7376x; 5.7376x over previous
import dataclasses
import functools

import jax
import jax.numpy as jnp
from jax import lax
from jax.experimental import pallas as pl
from jax.experimental.pallas import tpu as pltpu
from jax.experimental.pallas import tpu_sc as plsc

N = 10000
E = 320000
D = 128
L = 16

NC = 2
NS = 16
NW = NC * NS
EDGES_PER_W = E // NW
CHUNK = 80
IBLK = 2000
NBLK = EDGES_PER_W // IBLK
CPB = IBLK // CHUNK
N_PAD = 10240
ROWS_PER_SUB = N_PAD // NS


def _sc_side(ego, row3, col, vals):
    mesh = plsc.VectorSubcoreMesh(core_axis_name="c", subcore_axis_name="s")

    cp = pltpu.CompilerParams()
    if "needs_layout_passes" in pltpu.CompilerParams.__dataclass_fields__:
        cp = dataclasses.replace(cp, needs_layout_passes=False)

    @functools.partial(
        pl.kernel,
        compiler_params=cp,
        out_type=jax.ShapeDtypeStruct((NC, N_PAD, D), jnp.float32),
        mesh=mesh,
        scratch_types=[
            pltpu.VMEM((CPB, CHUNK), jnp.int32),
            pltpu.VMEM((IBLK,), jnp.int32),
            pltpu.VMEM((IBLK,), jnp.float32),
            pltpu.VMEM((CHUNK, D), jnp.float32),
            pltpu.VMEM_SHARED((N_PAD, D), jnp.float32),
            pltpu.SemaphoreType.DMA,
        ],
    )
    def body(ego_hbm, row_hbm, col_hbm, val_hbm, out_hbm,
             row_v, col_v, val_v, rows_v, acc_sh, sem):
        core = lax.axis_index("c")
        sid = lax.axis_index("s")
        wid = core * NS + sid
        base = wid * EDGES_PER_W

        zvec = jnp.zeros((L,), jnp.float32)

        @pl.loop(0, CHUNK)
        def _(i):
            for j in range(D // L):
                rows_v[i, pl.ds(j * L, L)] = zvec

        @pl.loop(0, ROWS_PER_SUB // CHUNK)
        def _(k):
            pltpu.sync_copy(
                rows_v,
                acc_sh.at[pl.ds(sid * ROWS_PER_SUB + k * CHUNK, CHUNK)])

        plsc.subcore_barrier()

        @pl.loop(0, NBLK)
        def _(bk):
            pltpu.sync_copy(row_hbm.at[wid, bk], row_v)
            pltpu.sync_copy(col_hbm.at[pl.ds(base + bk * IBLK, IBLK)], col_v)
            pltpu.sync_copy(val_hbm.at[pl.ds(base + bk * IBLK, IBLK)], val_v)

            @pl.loop(0, CPB)
            def _(c):
                pltpu.async_copy(
                    ego_hbm.at[col_v.at[pl.ds(c * CHUNK, CHUNK)]], rows_v,
                    sem).wait()

                @pl.loop(0, CHUNK)
                def _(i):
                    idx = jnp.full((L,), c * CHUNK + i, jnp.int32)
                    vv = plsc.load_gather(val_v, [idx])
                    for j in range(D // L):
                        sl = (i, pl.ds(j * L, L))
                        rows_v[sl] = rows_v[sl] * vv

                pltpu.sync_copy(rows_v, acc_sh.at[row_v.at[c]], add=True)

        plsc.subcore_barrier()

        pltpu.sync_copy(
            acc_sh.at[pl.ds(sid * ROWS_PER_SUB, ROWS_PER_SUB)],
            out_hbm.at[core, pl.ds(sid * ROWS_PER_SUB, ROWS_PER_SUB)])

    return body(ego, row3, col, vals)


def _tc_body(ego_b, s_b0, s_b1, w_b, bias_b, out_b):
    hi = ego_b[...] + s_b0[...][0] + s_b1[...][0]
    acc = lax.dot_general(hi, w_b[...], (((1,), (1,)), ((), ())),
                          preferred_element_type=jnp.float32)
    acc = acc + bias_b[...]
    out_b[...] = jnp.where(acc >= 0, acc, 0.01 * acc)


def _tc_linear(ego, partials, W, bias2d):
    blk = 1000
    return pl.pallas_call(
        _tc_body,
        grid=(N // blk,),
        in_specs=[
            pl.BlockSpec((blk, D), lambda i: (i, 0)),
            pl.BlockSpec((1, blk, D), lambda i: (0, i, 0)),
            pl.BlockSpec((1, blk, D), lambda i: (1, i, 0)),
            pl.BlockSpec((D, D), lambda i: (0, 0)),
            pl.BlockSpec((1, D), lambda i: (0, 0)),
        ],
        out_specs=pl.BlockSpec((blk, D), lambda i: (i, 0)),
        out_shape=jax.ShapeDtypeStruct((N, D), jnp.float32),
    )(ego, partials, partials, W, bias2d)


def kernel(ego_embeddings, edge_index, edge_vals, h0, W, b, lamda, alpha, l):
    row = edge_index[0].reshape(NW, NBLK, CPB, CHUNK)
    col = edge_index[1]
    partials = _sc_side(ego_embeddings, row, col, edge_vals)
    return _tc_linear(ego_embeddings, partials, W, b.reshape(1, D))


# speedup vs baseline: 9.2588x; 1.6137x over previous
import dataclasses
import functools

import jax
import jax.numpy as jnp
from jax import lax
from jax.experimental import pallas as pl
from jax.experimental.pallas import tpu as pltpu
from jax.experimental.pallas import tpu_sc as plsc

N = 10000
E = 320000
D = 128
L = 16

NC = 2
NS = 16
NW = NC * NS
EDGES_PER_W = E // NW
CHUNK = 80
IBLK = 2000
NBLK = EDGES_PER_W // IBLK
CPB = IBLK // CHUNK
N_PAD = 10240
ROWS_PER_SUB = N_PAD // NS


def _sc_side(ego, row3, col, vals):
    mesh = plsc.VectorSubcoreMesh(core_axis_name="c", subcore_axis_name="s")

    cp = pltpu.CompilerParams()
    if "needs_layout_passes" in pltpu.CompilerParams.__dataclass_fields__:
        cp = dataclasses.replace(cp, needs_layout_passes=False)

    @functools.partial(
        pl.kernel,
        compiler_params=cp,
        out_type=jax.ShapeDtypeStruct((NC, N_PAD, D), jnp.float32),
        mesh=mesh,
        scratch_types=[
            pltpu.VMEM((CPB, CHUNK), jnp.int32),
            pltpu.VMEM((IBLK,), jnp.int32),
            pltpu.VMEM((IBLK,), jnp.float32),
            pltpu.VMEM((CHUNK, D), jnp.float32),
            pltpu.VMEM((CHUNK, D), jnp.float32),
            pltpu.VMEM_SHARED((N_PAD, D), jnp.float32),
            pltpu.SemaphoreType.DMA,
            pltpu.SemaphoreType.DMA,
        ],
    )
    def body(ego_hbm, row_hbm, col_hbm, val_hbm, out_hbm,
             row_v, col_v, val_v, rows_a, rows_b, acc_sh, sem_a, sem_b):
        core = lax.axis_index("c")
        sid = lax.axis_index("s")
        wid = core * NS + sid
        base = wid * EDGES_PER_W

        zvec = jnp.zeros((L,), jnp.float32)

        @pl.loop(0, CHUNK)
        def _(i):
            for j in range(D // L):
                rows_a[i, pl.ds(j * L, L)] = zvec

        @pl.loop(0, ROWS_PER_SUB // CHUNK)
        def _(k):
            pltpu.sync_copy(
                rows_a,
                acc_sh.at[pl.ds(sid * ROWS_PER_SUB + k * CHUNK, CHUNK)])

        plsc.subcore_barrier()

        def g_start(c, buf, sem):
            pltpu.async_copy(
                ego_hbm.at[col_v.at[pl.ds(c * CHUNK, CHUNK)]], buf, sem)

        def g_wait(buf, sem):
            pltpu.make_async_copy(
                ego_hbm.at[col_v.at[pl.ds(0, CHUNK)]], buf, sem).wait()

        def scale_scatter(c, buf):
            @pl.loop(0, CHUNK, step=2)
            def _(i):
                for u in range(2):
                    idx = jnp.full((L,), c * CHUNK + i + u, jnp.int32)
                    vv = plsc.load_gather(val_v, [idx])
                    for j in range(D // L):
                        sl = (i + u, pl.ds(j * L, L))
                        buf[sl] = buf[sl] * vv

            pltpu.sync_copy(buf, acc_sh.at[row_v.at[c]], add=True)

        @pl.loop(0, NBLK)
        def _(bk):
            pltpu.sync_copy(row_hbm.at[wid, bk], row_v)
            pltpu.sync_copy(col_hbm.at[pl.ds(base + bk * IBLK, IBLK)], col_v)
            pltpu.sync_copy(val_hbm.at[pl.ds(base + bk * IBLK, IBLK)], val_v)

            g_start(0, rows_a, sem_a)

            @pl.loop(0, (CPB - 1) // 2)
            def _(p):
                c0 = 2 * p
                g_wait(rows_a, sem_a)
                g_start(c0 + 1, rows_b, sem_b)
                scale_scatter(c0, rows_a)
                g_wait(rows_b, sem_b)
                g_start(c0 + 2, rows_a, sem_a)
                scale_scatter(c0 + 1, rows_b)

            g_wait(rows_a, sem_a)
            scale_scatter(CPB - 1, rows_a)

        plsc.subcore_barrier()

        pltpu.sync_copy(
            acc_sh.at[pl.ds(sid * ROWS_PER_SUB, ROWS_PER_SUB)],
            out_hbm.at[core, pl.ds(sid * ROWS_PER_SUB, ROWS_PER_SUB)])

    return body(ego, row3, col, vals)


def _tc_body(ego_b, s_b0, s_b1, w_b, bias_b, out_b):
    hi = ego_b[...] + s_b0[...][0] + s_b1[...][0]
    acc = lax.dot_general(hi, w_b[...], (((1,), (1,)), ((), ())),
                          preferred_element_type=jnp.float32)
    acc = acc + bias_b[...]
    out_b[...] = jnp.where(acc >= 0, acc, 0.01 * acc)


def _tc_linear(ego, partials, W, bias2d):
    blk = 1000
    return pl.pallas_call(
        _tc_body,
        grid=(N // blk,),
        in_specs=[
            pl.BlockSpec((blk, D), lambda i: (i, 0)),
            pl.BlockSpec((1, blk, D), lambda i: (0, i, 0)),
            pl.BlockSpec((1, blk, D), lambda i: (1, i, 0)),
            pl.BlockSpec((D, D), lambda i: (0, 0)),
            pl.BlockSpec((1, D), lambda i: (0, 0)),
        ],
        out_specs=pl.BlockSpec((blk, D), lambda i: (i, 0)),
        out_shape=jax.ShapeDtypeStruct((N, D), jnp.float32),
    )(ego, partials, partials, W, bias2d)


def kernel(ego_embeddings, edge_index, edge_vals, h0, W, b, lamda, alpha, l):
    row = edge_index[0].reshape(NW, NBLK, CPB, CHUNK)
    col = edge_index[1]
    partials = _sc_side(ego_embeddings, row, col, edge_vals)
    return _tc_linear(ego_embeddings, partials, W, b.reshape(1, D))


# speedup vs baseline: 10.4305x; 1.1265x over previous
import dataclasses
import functools

import jax
import jax.numpy as jnp
from jax import lax
from jax.experimental import pallas as pl
from jax.experimental.pallas import tpu as pltpu
from jax.experimental.pallas import tpu_sc as plsc

N = 10000
E = 320000
D = 128
L = 16

NC = 2
NS = 16
NW = NC * NS
EDGES_PER_W = E // NW
CHUNK = 80
IBLK = 2000
NBLK = EDGES_PER_W // IBLK
CPB = IBLK // CHUNK
N_PAD = 10240
ROWS_PER_SUB = N_PAD // NS


def _sc_side(ego, row3, col, vals):
    mesh = plsc.VectorSubcoreMesh(core_axis_name="c", subcore_axis_name="s")

    cp = pltpu.CompilerParams()
    if "needs_layout_passes" in pltpu.CompilerParams.__dataclass_fields__:
        cp = dataclasses.replace(cp, needs_layout_passes=False)

    @functools.partial(
        pl.kernel,
        compiler_params=cp,
        out_type=jax.ShapeDtypeStruct((NC, N_PAD, D), jnp.float32),
        mesh=mesh,
        scratch_types=[
            pltpu.VMEM((CPB, CHUNK), jnp.int32),
            pltpu.VMEM((IBLK,), jnp.int32),
            pltpu.VMEM((IBLK,), jnp.float32),
            pltpu.VMEM((CHUNK, D), jnp.float32),
            pltpu.VMEM((CHUNK, D), jnp.float32),
            pltpu.VMEM((CHUNK, D), jnp.float32),
            pltpu.VMEM_SHARED((N_PAD, D), jnp.float32),
            pltpu.SemaphoreType.DMA,
            pltpu.SemaphoreType.DMA,
            pltpu.SemaphoreType.DMA,
            pltpu.SemaphoreType.DMA,
            pltpu.SemaphoreType.DMA,
            pltpu.SemaphoreType.DMA,
        ],
    )
    def body(ego_hbm, row_hbm, col_hbm, val_hbm, out_hbm,
             row_v, col_v, val_v, rows_0, rows_1, rows_2, acc_sh,
             gs0, gs1, gs2, ss0, ss1, ss2):
        core = lax.axis_index("c")
        sid = lax.axis_index("s")
        wid = core * NS + sid
        base = wid * EDGES_PER_W

        zvec = jnp.zeros((L,), jnp.float32)

        @pl.loop(0, CHUNK)
        def _(i):
            for j in range(D // L):
                rows_0[i, pl.ds(j * L, L)] = zvec

        @pl.loop(0, ROWS_PER_SUB // CHUNK)
        def _(k):
            pltpu.sync_copy(
                rows_0,
                acc_sh.at[pl.ds(sid * ROWS_PER_SUB + k * CHUNK, CHUNK)])

        plsc.subcore_barrier()

        bufs = ((rows_0, gs0, ss0), (rows_1, gs1, ss1), (rows_2, gs2, ss2))

        def g_start(c, k):
            pltpu.async_copy(
                ego_hbm.at[col_v.at[pl.ds(c * CHUNK, CHUNK)]],
                bufs[k][0], bufs[k][1])

        def g_wait(k):
            pltpu.make_async_copy(
                ego_hbm.at[col_v.at[pl.ds(0, CHUNK)]],
                bufs[k][0], bufs[k][1]).wait()

        def s_start(c, k):
            pltpu.async_copy(bufs[k][0], acc_sh.at[row_v.at[c]], bufs[k][2],
                             add=True)

        def s_wait(k):
            pltpu.make_async_copy(bufs[k][0], acc_sh.at[row_v.at[0]],
                                  bufs[k][2]).wait()

        def scale(c, k):
            buf = bufs[k][0]

            @pl.loop(0, CHUNK, step=2)
            def _(i):
                for u in range(2):
                    idx = jnp.full((L,), c * CHUNK + i + u, jnp.int32)
                    vv = plsc.load_gather(val_v, [idx])
                    for j in range(D // L):
                        sl = (i + u, pl.ds(j * L, L))
                        buf[sl] = buf[sl] * vv

        def step(c, k, swait, gstart):
            k1 = (k + 1) % 3
            if swait:
                s_wait(k1)
            if gstart:
                g_start(c + 1, k1)
            g_wait(k)
            scale(c, k)
            s_start(c, k)

        @pl.loop(0, NBLK)
        def _(bk):
            pltpu.sync_copy(row_hbm.at[wid, bk], row_v)
            pltpu.sync_copy(col_hbm.at[pl.ds(base + bk * IBLK, IBLK)], col_v)
            pltpu.sync_copy(val_hbm.at[pl.ds(base + bk * IBLK, IBLK)], val_v)

            g_start(0, 0)
            g_start(1, 1)
            step(0, 0, swait=False, gstart=False)
            g_start(2, 2)
            step(1, 1, swait=False, gstart=False)

            @pl.loop(0, (CPB - 4) // 3)
            def _(t):
                c0 = 3 * t + 2
                step(c0, 2, swait=True, gstart=True)
                step(c0 + 1, 0, swait=True, gstart=True)
                step(c0 + 2, 1, swait=True, gstart=True)

            step(CPB - 2, 2, swait=True, gstart=True)
            step(CPB - 1, 0, swait=True, gstart=False)
            s_wait(2)
            s_wait(0)

        plsc.subcore_barrier()

        pltpu.sync_copy(
            acc_sh.at[pl.ds(sid * ROWS_PER_SUB, ROWS_PER_SUB)],
            out_hbm.at[core, pl.ds(sid * ROWS_PER_SUB, ROWS_PER_SUB)])

    return body(ego, row3, col, vals)


def _tc_body(ego_b, s_b0, s_b1, w_b, bias_b, out_b):
    hi = ego_b[...] + s_b0[...][0] + s_b1[...][0]
    acc = lax.dot_general(hi, w_b[...], (((1,), (1,)), ((), ())),
                          preferred_element_type=jnp.float32)
    acc = acc + bias_b[...]
    out_b[...] = jnp.where(acc >= 0, acc, 0.01 * acc)


def _tc_linear(ego, partials, W, bias2d):
    blk = 1000
    return pl.pallas_call(
        _tc_body,
        grid=(N // blk,),
        in_specs=[
            pl.BlockSpec((blk, D), lambda i: (i, 0)),
            pl.BlockSpec((1, blk, D), lambda i: (0, i, 0)),
            pl.BlockSpec((1, blk, D), lambda i: (1, i, 0)),
            pl.BlockSpec((D, D), lambda i: (0, 0)),
            pl.BlockSpec((1, D), lambda i: (0, 0)),
        ],
        out_specs=pl.BlockSpec((blk, D), lambda i: (i, 0)),
        out_shape=jax.ShapeDtypeStruct((N, D), jnp.float32),
    )(ego, partials, partials, W, bias2d)


def kernel(ego_embeddings, edge_index, edge_vals, h0, W, b, lamda, alpha, l):
    row = edge_index[0].reshape(NW, NBLK, CPB, CHUNK)
    col = edge_index[1]
    partials = _sc_side(ego_embeddings, row, col, edge_vals)
    return _tc_linear(ego_embeddings, partials, W, b.reshape(1, D))
